# baseline (speedup 1.0000x reference)
import functools

import jax
import jax.numpy as jnp
from jax import lax
from jax.experimental import pallas as pl
from jax.experimental.pallas import tpu as pltpu
from jax.experimental.pallas import tpu_sc as plsc

N = 100000
D = 128
G = 64
BLK = 20000
NBLK = N // BLK

_NC = 2
_NS = 16
ROWS_PER_TILE = G // (_NC * _NS)
_VPR = D // 16


def _tc_body(nodes_ref, gid_ref, w_ref, b_ref, part_ref):
    x = nodes_ref[...]
    pre = lax.dot_general(x, w_ref[...], (((1,), (1,)), ((), ())),
                          preferred_element_type=jnp.float32)
    pre = pre + b_ref[...]
    scale = 1.0507009873554804934193349852946
    alpha = 1.6732632423543772848170429916717
    neg = alpha * (jnp.exp(jnp.minimum(pre, 0.0)) - 1.0)
    act = scale * jnp.where(pre > 0, pre, neg)

    gid = gid_ref[0, 0, :]
    cols = lax.broadcasted_iota(jnp.int32, (BLK, G), 1)
    onehot = (gid[:, None] == cols).astype(jnp.float32)
    part_ref[0] = lax.dot_general(onehot, act, (((0,), (0,)), ((), ())),
                                  preferred_element_type=jnp.float32)


def _sc_body(part_hbm, out_hbm, slab, obuf):
    w = lax.axis_index("s") * _NC + lax.axis_index("c")
    r0 = w * ROWS_PER_TILE
    pltpu.sync_copy(part_hbm.at[:, pl.ds(r0, ROWS_PER_TILE), :], slab)

    nv = ROWS_PER_TILE * _VPR

    def body(blk, acc):
        return tuple(
            acc[k] + slab[blk, k // _VPR, pl.ds((k % _VPR) * 16, 16)]
            for k in range(nv))

    acc = lax.fori_loop(
        0, NBLK, body,
        tuple(jnp.zeros((16,), jnp.float32) for _ in range(nv)))

    for k in range(nv):
        a = acc[k]
        t = 1.0 - 2.0 / (jnp.exp(2.0 * a) + 1.0)
        obuf[k // _VPR, pl.ds((k % _VPR) * 16, 16)] = t

    pltpu.sync_copy(obuf, out_hbm.at[pl.ds(r0, ROWS_PER_TILE), :])


@jax.jit
def kernel(nodes, graph_id, W, b):
    gid3 = graph_id.reshape(NBLK, 1, BLK)
    partial = pl.pallas_call(
        _tc_body,
        grid=(NBLK,),
        in_specs=[
            pl.BlockSpec((BLK, D), lambda i: (i, 0)),
            pl.BlockSpec((1, 1, BLK), lambda i: (i, 0, 0)),
            pl.BlockSpec((D, D), lambda i: (0, 0)),
            pl.BlockSpec((1, D), lambda i: (0, 0)),
        ],
        out_specs=pl.BlockSpec((1, G, D), lambda i: (i, 0, 0)),
        out_shape=jax.ShapeDtypeStruct((NBLK, G, D), jnp.float32),
    )(nodes, gid3, W, b.reshape(1, D))

    mesh = plsc.VectorSubcoreMesh(core_axis_name="c", subcore_axis_name="s")
    sc_reduce = functools.partial(
        pl.kernel,
        mesh=mesh,
        out_type=jax.ShapeDtypeStruct((G, D), jnp.float32),
        scratch_types=[
            pltpu.VMEM((NBLK, ROWS_PER_TILE, D), jnp.float32),
            pltpu.VMEM((ROWS_PER_TILE, D), jnp.float32),
        ],
    )(_sc_body)
    return sc_reduce(partial)


# speedup vs baseline: 1.0264x; 1.0264x over previous
import functools

import jax
import jax.numpy as jnp
from jax import lax
from jax.experimental import pallas as pl
from jax.experimental.pallas import tpu as pltpu
from jax.experimental.pallas import tpu_sc as plsc

N = 100000
D = 128
G = 64
BLK = 10000
NBLK = N // BLK

_NC = 2
_NS = 16
ROWS_PER_TILE = G // (_NC * _NS)
_VPR = D // 16


def _tc_body(nodes_ref, gid_ref, w_ref, b_ref, sum_ref, acc_ref):
    i = pl.program_id(0)

    x = nodes_ref[...]
    pre = lax.dot_general(x, w_ref[...], (((1,), (1,)), ((), ())),
                          preferred_element_type=jnp.float32)
    pre = pre + b_ref[...]
    scale = 1.0507009873554804934193349852946
    alpha = 1.6732632423543772848170429916717
    neg = alpha * (jnp.exp(jnp.minimum(pre, 0.0)) - 1.0)
    act = scale * jnp.where(pre > 0, pre, neg)

    gid = gid_ref[0, 0, :]
    cols = lax.broadcasted_iota(jnp.int32, (BLK, G), 1)
    onehot = (gid[:, None] == cols).astype(jnp.float32)
    part = lax.dot_general(onehot, act, (((0,), (0,)), ((), ())),
                           preferred_element_type=jnp.float32)

    @pl.when(i == 0)
    def _init():
        acc_ref[...] = jnp.zeros_like(acc_ref)

    acc_ref[...] += part

    @pl.when(i == NBLK - 1)
    def _fin():
        sum_ref[...] = acc_ref[...]


def _sc_body(sum_hbm, out_hbm, buf):
    w = lax.axis_index("s") * _NC + lax.axis_index("c")
    r0 = w * ROWS_PER_TILE
    pltpu.sync_copy(sum_hbm.at[pl.ds(r0, ROWS_PER_TILE), :], buf)

    for k in range(ROWS_PER_TILE * _VPR):
        sl = (k // _VPR, pl.ds((k % _VPR) * 16, 16))
        a = buf[sl]
        buf[sl] = 1.0 - 2.0 / (jnp.exp(2.0 * a) + 1.0)

    pltpu.sync_copy(buf, out_hbm.at[pl.ds(r0, ROWS_PER_TILE), :])


@jax.jit
def kernel(nodes, graph_id, W, b):
    gid3 = graph_id.reshape(NBLK, 1, BLK)
    sums = pl.pallas_call(
        _tc_body,
        grid=(NBLK,),
        in_specs=[
            pl.BlockSpec((BLK, D), lambda i: (i, 0)),
            pl.BlockSpec((1, 1, BLK), lambda i: (i, 0, 0)),
            pl.BlockSpec((D, D), lambda i: (0, 0)),
            pl.BlockSpec((1, D), lambda i: (0, 0)),
        ],
        out_specs=pl.BlockSpec((G, D), lambda i: (0, 0)),
        out_shape=jax.ShapeDtypeStruct((G, D), jnp.float32),
        scratch_shapes=[pltpu.VMEM((G, D), jnp.float32)],
    )(nodes, gid3, W, b.reshape(1, D))

    mesh = plsc.VectorSubcoreMesh(core_axis_name="c", subcore_axis_name="s")
    sc_tanh = functools.partial(
        pl.kernel,
        mesh=mesh,
        out_type=jax.ShapeDtypeStruct((G, D), jnp.float32),
        scratch_types=[
            pltpu.VMEM((ROWS_PER_TILE, D), jnp.float32),
        ],
    )(_sc_body)
    return sc_tanh(sums)
